# Initial kernel scaffold; baseline (speedup 1.0000x reference)
import jax
import jax.numpy as jnp
from jax.experimental import pallas as pl


def kernel(x, W0, b0, W1, b1, hyperedges):
    raise NotImplementedError("write your pallas kernel here")


# speedup vs baseline: 21.4402x; 21.4402x over previous
import functools

import jax
import jax.numpy as jnp
from jax import lax
from jax.experimental import pallas as pl
from jax.experimental.pallas import tpu as pltpu
from jax.experimental.pallas import tpu_sc as plsc

N = 100000
NP = 100352
EH = 100000
EH_PAD = 102400
K = 16
NW = 32
EPT = EH_PAD // NW
CHUNK = 128
NCHUNK = EPT // CHUNK
TSLAB = NP // 16
ROWBLK = 2048

_MESH = plsc.VectorSubcoreMesh(core_axis_name="c", subcore_axis_name="s")
_SC_PARAMS = pltpu.CompilerParams(needs_layout_passes=False,
                                  use_tc_tiling_on_sc=False)


def _proj_body(x_ref, w_ref, rv_ref, m_ref, p_ref):
    xb = x_ref[...].astype(jnp.bfloat16)
    m_ref[...] = jnp.dot(xb, w_ref[...].astype(jnp.bfloat16),
                         preferred_element_type=jnp.float32)
    p_ref[...] = jnp.dot(xb, rv_ref[...].astype(jnp.bfloat16),
                         preferred_element_type=jnp.float32)


def _tc_project(hx, w, rv):
    din = hx.shape[1]
    dout = w.shape[1]
    return pl.pallas_call(
        _proj_body,
        grid=(NP // ROWBLK,),
        in_specs=[
            pl.BlockSpec((ROWBLK, din), lambda i: (i, 0)),
            pl.BlockSpec((din, dout), lambda i: (0, 0)),
            pl.BlockSpec((din, 1), lambda i: (0, 0)),
        ],
        out_specs=[
            pl.BlockSpec((ROWBLK, dout), lambda i: (i, 0)),
            pl.BlockSpec((ROWBLK, 1), lambda i: (i, 0)),
        ],
        out_shape=[
            jax.ShapeDtypeStruct((NP, dout), jnp.float32),
            jax.ShapeDtypeStruct((NP, 1), jnp.float32),
        ],
    )(hx, w, rv)


def _norm_body(dega_ref, degb_ref, m_ref, dinv_ref, mn_ref):
    deg = 1.0 + dega_ref[...] + degb_ref[...]
    dv = lax.rsqrt(deg)
    dvc = dv.reshape(ROWBLK, 1)
    dinv_ref[...] = jnp.broadcast_to(dvc, (ROWBLK, 16))
    mn_ref[...] = m_ref[...] * dvc


def _tc_norm(dega, degb, m):
    spec = pl.BlockSpec((ROWBLK, 16), lambda i: (i, 0))
    dspec = pl.BlockSpec((1, ROWBLK), lambda i: (0, i))
    return pl.pallas_call(
        _norm_body,
        grid=(NP // ROWBLK,),
        in_specs=[dspec, dspec, spec],
        out_specs=[spec, spec],
        out_shape=[
            jax.ShapeDtypeStruct((NP, 16), jnp.float32),
            jax.ShapeDtypeStruct((NP, 16), jnp.float32),
        ],
    )(dega, degb, m)


def _out_body(dinv_ref, agga_ref, aggb_ref, m_ref, b_ref, o_ref):
    dv = dinv_ref[...]
    agg = agga_ref[...] + aggb_ref[...]
    o = dv * agg + (dv * dv) * m_ref[...] + b_ref[...]
    o_ref[...] = jnp.maximum(o, 0.0)


def _tc_out(dinv, agga, aggb, m, b2d):
    spec = pl.BlockSpec((ROWBLK, 16), lambda i: (i, 0))
    return pl.pallas_call(
        _out_body,
        grid=(NP // ROWBLK,),
        in_specs=[spec, spec, spec, spec,
                  pl.BlockSpec((1, 16), lambda i: (0, 0))],
        out_specs=spec,
        out_shape=jax.ShapeDtypeStruct((NP, 16), jnp.float32),
    )(dinv, agga, aggb, m, b2d)


@functools.partial(
    pl.kernel,
    out_type=[
        jax.ShapeDtypeStruct((EH_PAD,), jnp.int32),
        jax.ShapeDtypeStruct((EH_PAD,), jnp.int32),
        jax.ShapeDtypeStruct((2, NP), jnp.float32),
    ],
    mesh=_MESH,
    compiler_params=_SC_PARAMS,
    scratch_types=[
        pltpu.VMEM((NP,), jnp.float32),
        pltpu.VMEM((K, CHUNK), jnp.int32),
        pltpu.VMEM((CHUNK,), jnp.int32),
        pltpu.VMEM((CHUNK,), jnp.int32),
        pltpu.VMEM((CHUNK,), jnp.float32),
        pltpu.VMEM_SHARED((NP,), jnp.float32),
        pltpu.SemaphoreType.DMA,
    ],
)
def _sc_edges(p_hbm, het_hbm, z1_hbm, ones_hbm,
              se_hbm, ie_hbm, deg_hbm,
              p_v, he_v, se_v, ie_v, ones_v, deg_sp, sem):
    ci = lax.axis_index("c")
    si = lax.axis_index("s")
    wid = si * 2 + ci

    pltpu.sync_copy(p_hbm, p_v)
    pltpu.sync_copy(ones_hbm, ones_v)
    pltpu.sync_copy(z1_hbm.at[pl.ds(si * TSLAB, TSLAB)],
                    deg_sp.at[pl.ds(si * TSLAB, TSLAB)])
    plsc.subcore_barrier()

    def chunk_body(c, _):
        base = wid * EPT + c * CHUNK
        pltpu.sync_copy(het_hbm.at[:, pl.ds(base, CHUNK)], he_v)

        def group_body(j, _):
            he0 = he_v[0, pl.ds(j * 16, 16)]
            pe0 = plsc.load_gather(p_v, [he0])
            cmax = pe0
            cmin = pe0
            se = he0
            ie = he0
            for k in range(1, K):
                hek = he_v[k, pl.ds(j * 16, 16)]
                pek = plsc.load_gather(p_v, [hek])
                gt = pek > cmax
                lt = pek < cmin
                se = jnp.where(gt, hek, se)
                cmax = jnp.where(gt, pek, cmax)
                ie = jnp.where(lt, hek, ie)
                cmin = jnp.where(lt, pek, cmin)
            se_v[pl.ds(j * 16, 16)] = se
            ie_v[pl.ds(j * 16, 16)] = ie
            return 0

        lax.fori_loop(0, CHUNK // 16, group_body, 0)
        pltpu.sync_copy(se_v, se_hbm.at[pl.ds(base, CHUNK)])
        pltpu.sync_copy(ie_v, ie_hbm.at[pl.ds(base, CHUNK)])
        pltpu.sync_copy(ones_v, deg_sp.at[se_v], add=True)
        pltpu.sync_copy(ones_v, deg_sp.at[ie_v], add=True)
        return 0

    lax.fori_loop(0, NCHUNK, chunk_body, 0)
    plsc.subcore_barrier()
    pltpu.sync_copy(deg_sp.at[pl.ds(si * TSLAB, TSLAB)],
                    deg_hbm.at[ci, pl.ds(si * TSLAB, TSLAB)])


@functools.partial(
    pl.kernel,
    out_type=jax.ShapeDtypeStruct((2, NP, 16), jnp.float32),
    mesh=_MESH,
    compiler_params=_SC_PARAMS,
    scratch_types=[
        pltpu.VMEM((CHUNK,), jnp.int32),
        pltpu.VMEM((CHUNK,), jnp.int32),
        pltpu.VMEM((CHUNK, 16), jnp.float32),
        pltpu.VMEM((CHUNK, 16), jnp.float32),
        pltpu.VMEM_SHARED((NP, 16), jnp.float32),
        pltpu.SemaphoreType.DMA,
    ],
)
def _sc_agg(se_hbm, ie_hbm, mn_hbm, z2_hbm,
            agg_hbm,
            se_v, ie_v, rows_a, rows_b, agg_sp, sem):
    ci = lax.axis_index("c")
    si = lax.axis_index("s")
    wid = si * 2 + ci

    pltpu.sync_copy(z2_hbm.at[pl.ds(si * TSLAB, TSLAB)],
                    agg_sp.at[pl.ds(si * TSLAB, TSLAB)])
    plsc.subcore_barrier()

    def chunk_body(c, _):
        base = wid * EPT + c * CHUNK
        pltpu.sync_copy(se_hbm.at[pl.ds(base, CHUNK)], se_v)
        pltpu.sync_copy(ie_hbm.at[pl.ds(base, CHUNK)], ie_v)
        pltpu.async_copy(mn_hbm.at[ie_v], rows_a, sem).wait()
        pltpu.async_copy(mn_hbm.at[se_v], rows_b, sem).wait()
        pltpu.sync_copy(rows_a, agg_sp.at[se_v], add=True)
        pltpu.sync_copy(rows_b, agg_sp.at[ie_v], add=True)
        return 0

    lax.fori_loop(0, NCHUNK, chunk_body, 0)
    plsc.subcore_barrier()
    pltpu.sync_copy(agg_sp.at[pl.ds(si * TSLAB, TSLAB)],
                    agg_hbm.at[ci, pl.ds(si * TSLAB, TSLAB)])


def _layer(hx_pad, w, b, rv, het, z2, ones2):
    m, p2 = _tc_project(hx_pad, w, rv[:, None])
    p = p2.reshape(NP)
    se, ie, deg2 = _sc_edges(p, het, z2[:, 0], ones2)
    dinv, mn = _tc_norm(deg2[0].reshape(1, NP), deg2[1].reshape(1, NP), m)
    agg2 = _sc_agg(se, ie, mn, z2)
    return _tc_out(dinv, agg2[0], agg2[1], m, b.reshape(1, 16))


def kernel(x, W0, b0, W1, b1, hyperedges):
    rv_key = jax.random.key(1)
    rv0 = jax.random.uniform(jax.random.fold_in(rv_key, 0), (128,),
                             dtype=jnp.float32)
    rv1 = jax.random.uniform(jax.random.fold_in(rv_key, 1), (16,),
                             dtype=jnp.float32)
    xp = jnp.pad(x, ((0, NP - N), (0, 0)))
    het = jnp.pad(hyperedges.astype(jnp.int32), ((0, EH_PAD - EH), (0, 0)),
                  constant_values=N).T
    z2 = jnp.zeros((NP, 16), jnp.float32)
    ones2 = jnp.ones((CHUNK,), jnp.float32)
    h = _layer(xp, W0, b0, rv0, het, z2, ones2)
    h = _layer(h, W1, b1, rv1, het, z2, ones2)
    return h[:N]


# speedup vs baseline: 21.8125x; 1.0174x over previous
import functools

import jax
import jax.numpy as jnp
from jax import lax
from jax.experimental import pallas as pl
from jax.experimental.pallas import tpu as pltpu
from jax.experimental.pallas import tpu_sc as plsc

N = 100000
NP = 100352
EH = 100000
EH_PAD = 102400
K = 16
NW = 32
EPT = EH_PAD // NW
CHUNK = 128
NCHUNK = EPT // CHUNK
TSLAB = NP // 16
ROWBLK = 2048

_MESH = plsc.VectorSubcoreMesh(core_axis_name="c", subcore_axis_name="s")
_SC_PARAMS = pltpu.CompilerParams(needs_layout_passes=False,
                                  use_tc_tiling_on_sc=False)


def _proj_body(x_ref, w_ref, rv_ref, m_ref, p_ref):
    xb = x_ref[...].astype(jnp.bfloat16)
    m_ref[...] = jnp.dot(xb, w_ref[...].astype(jnp.bfloat16),
                         preferred_element_type=jnp.float32)
    p_ref[...] = jnp.dot(xb, rv_ref[...].astype(jnp.bfloat16),
                         preferred_element_type=jnp.float32)


def _tc_project(hx, w, rv):
    din = hx.shape[1]
    dout = w.shape[1]
    return pl.pallas_call(
        _proj_body,
        grid=(NP // ROWBLK,),
        in_specs=[
            pl.BlockSpec((ROWBLK, din), lambda i: (i, 0)),
            pl.BlockSpec((din, dout), lambda i: (0, 0)),
            pl.BlockSpec((din, 1), lambda i: (0, 0)),
        ],
        out_specs=[
            pl.BlockSpec((ROWBLK, dout), lambda i: (i, 0)),
            pl.BlockSpec((ROWBLK, 1), lambda i: (i, 0)),
        ],
        out_shape=[
            jax.ShapeDtypeStruct((NP, dout), jnp.float32),
            jax.ShapeDtypeStruct((NP, 1), jnp.float32),
        ],
    )(hx, w, rv)


def _norm_body(dega_ref, degb_ref, m_ref, dinv_ref, mn_ref):
    deg = 1.0 + dega_ref[...] + degb_ref[...]
    dv = lax.rsqrt(deg)
    dvc = dv.reshape(ROWBLK, 1)
    dinv_ref[...] = jnp.broadcast_to(dvc, (ROWBLK, 16))
    mn_ref[...] = m_ref[...] * dvc


def _tc_norm(dega, degb, m):
    spec = pl.BlockSpec((ROWBLK, 16), lambda i: (i, 0))
    dspec = pl.BlockSpec((1, ROWBLK), lambda i: (0, i))
    return pl.pallas_call(
        _norm_body,
        grid=(NP // ROWBLK,),
        in_specs=[dspec, dspec, spec],
        out_specs=[spec, spec],
        out_shape=[
            jax.ShapeDtypeStruct((NP, 16), jnp.float32),
            jax.ShapeDtypeStruct((NP, 16), jnp.float32),
        ],
    )(dega, degb, m)


def _out_body(dinv_ref, agga_ref, aggb_ref, m_ref, b_ref, o_ref):
    dv = dinv_ref[...]
    agg = agga_ref[...] + aggb_ref[...]
    o = dv * agg + (dv * dv) * m_ref[...] + b_ref[...]
    o_ref[...] = jnp.maximum(o, 0.0)


def _tc_out(dinv, agga, aggb, m, b2d):
    spec = pl.BlockSpec((ROWBLK, 16), lambda i: (i, 0))
    return pl.pallas_call(
        _out_body,
        grid=(NP // ROWBLK,),
        in_specs=[spec, spec, spec, spec,
                  pl.BlockSpec((1, 16), lambda i: (0, 0))],
        out_specs=spec,
        out_shape=jax.ShapeDtypeStruct((NP, 16), jnp.float32),
    )(dinv, agga, aggb, m, b2d)


@functools.partial(
    pl.kernel,
    out_type=[
        jax.ShapeDtypeStruct((EH_PAD,), jnp.int32),
        jax.ShapeDtypeStruct((EH_PAD,), jnp.int32),
        jax.ShapeDtypeStruct((2, NP), jnp.float32),
    ],
    mesh=_MESH,
    compiler_params=_SC_PARAMS,
    scratch_types=[
        pltpu.VMEM((NP,), jnp.float32),
        pltpu.VMEM((K, CHUNK), jnp.int32),
        pltpu.VMEM((CHUNK,), jnp.int32),
        pltpu.VMEM((CHUNK,), jnp.int32),
        pltpu.VMEM((CHUNK,), jnp.float32),
        pltpu.VMEM_SHARED((NP,), jnp.float32),
        pltpu.SemaphoreType.DMA,
    ],
)
def _sc_edges(p_hbm, het_hbm, z1_hbm, ones_hbm,
              se_hbm, ie_hbm, deg_hbm,
              p_v, he_v, se_v, ie_v, ones_v, deg_sp, sem):
    ci = lax.axis_index("c")
    si = lax.axis_index("s")
    wid = si * 2 + ci

    pltpu.sync_copy(p_hbm, p_v)
    pltpu.sync_copy(ones_hbm, ones_v)
    pltpu.sync_copy(z1_hbm.at[pl.ds(si * TSLAB, TSLAB)],
                    deg_sp.at[pl.ds(si * TSLAB, TSLAB)])
    plsc.subcore_barrier()

    def chunk_body(c, _):
        base = wid * EPT + c * CHUNK
        pltpu.sync_copy(het_hbm.at[:, pl.ds(base, CHUNK)], he_v)

        def group_body(j, _):
            he0 = he_v[0, pl.ds(j * 16, 16)]
            pe0 = plsc.load_gather(p_v, [he0])
            cmax = pe0
            cmin = pe0
            se = he0
            ie = he0
            for k in range(1, K):
                hek = he_v[k, pl.ds(j * 16, 16)]
                pek = plsc.load_gather(p_v, [hek])
                gt = pek > cmax
                lt = pek < cmin
                se = jnp.where(gt, hek, se)
                cmax = jnp.where(gt, pek, cmax)
                ie = jnp.where(lt, hek, ie)
                cmin = jnp.where(lt, pek, cmin)
            se_v[pl.ds(j * 16, 16)] = se
            ie_v[pl.ds(j * 16, 16)] = ie
            return 0

        lax.fori_loop(0, CHUNK // 16, group_body, 0)
        pltpu.sync_copy(se_v, se_hbm.at[pl.ds(base, CHUNK)])
        pltpu.sync_copy(ie_v, ie_hbm.at[pl.ds(base, CHUNK)])
        pltpu.sync_copy(ones_v, deg_sp.at[se_v], add=True)
        pltpu.sync_copy(ones_v, deg_sp.at[ie_v], add=True)
        return 0

    lax.fori_loop(0, NCHUNK, chunk_body, 0)
    plsc.subcore_barrier()
    pltpu.sync_copy(deg_sp.at[pl.ds(si * TSLAB, TSLAB)],
                    deg_hbm.at[ci, pl.ds(si * TSLAB, TSLAB)])


@functools.partial(
    pl.kernel,
    out_type=jax.ShapeDtypeStruct((2, NP, 16), jnp.float32),
    mesh=_MESH,
    compiler_params=_SC_PARAMS,
    scratch_types=[
        pltpu.VMEM((CHUNK,), jnp.int32),
        pltpu.VMEM((CHUNK,), jnp.int32),
        pltpu.VMEM((CHUNK, 16), jnp.float32),
        pltpu.VMEM((CHUNK, 16), jnp.float32),
        pltpu.VMEM_SHARED((NP, 16), jnp.float32),
        pltpu.SemaphoreType.DMA,
    ],
)
def _sc_agg(se_hbm, ie_hbm, mn_hbm, z2_hbm,
            agg_hbm,
            se_v, ie_v, rows_a, rows_b, agg_sp, sem):
    ci = lax.axis_index("c")
    si = lax.axis_index("s")
    wid = si * 2 + ci

    pltpu.sync_copy(z2_hbm.at[pl.ds(si * TSLAB, TSLAB)],
                    agg_sp.at[pl.ds(si * TSLAB, TSLAB)])
    plsc.subcore_barrier()

    def chunk_body(c, _):
        base = wid * EPT + c * CHUNK
        pltpu.sync_copy(se_hbm.at[pl.ds(base, CHUNK)], se_v)
        pltpu.sync_copy(ie_hbm.at[pl.ds(base, CHUNK)], ie_v)
        ca = pltpu.async_copy(mn_hbm.at[ie_v], rows_a, sem)
        cb = pltpu.async_copy(mn_hbm.at[se_v], rows_b, sem)
        ca.wait()
        cb.wait()
        pltpu.sync_copy(rows_a, agg_sp.at[se_v], add=True)
        pltpu.sync_copy(rows_b, agg_sp.at[ie_v], add=True)
        return 0

    lax.fori_loop(0, NCHUNK, chunk_body, 0)
    plsc.subcore_barrier()
    pltpu.sync_copy(agg_sp.at[pl.ds(si * TSLAB, TSLAB)],
                    agg_hbm.at[ci, pl.ds(si * TSLAB, TSLAB)])


def _layer(hx_pad, w, b, rv, het, z2, ones2):
    m, p2 = _tc_project(hx_pad, w, rv[:, None])
    p = p2.reshape(NP)
    se, ie, deg2 = _sc_edges(p, het, z2[:, 0], ones2)
    dinv, mn = _tc_norm(deg2[0].reshape(1, NP), deg2[1].reshape(1, NP), m)
    agg2 = _sc_agg(se, ie, mn, z2)
    return _tc_out(dinv, agg2[0], agg2[1], m, b.reshape(1, 16))


def kernel(x, W0, b0, W1, b1, hyperedges):
    rv_key = jax.random.key(1)
    rv0 = jax.random.uniform(jax.random.fold_in(rv_key, 0), (128,),
                             dtype=jnp.float32)
    rv1 = jax.random.uniform(jax.random.fold_in(rv_key, 1), (16,),
                             dtype=jnp.float32)
    xp = jnp.pad(x, ((0, NP - N), (0, 0)))
    het = jnp.pad(hyperedges.astype(jnp.int32), ((0, EH_PAD - EH), (0, 0)),
                  constant_values=N).T
    z2 = jnp.zeros((NP, 16), jnp.float32)
    ones2 = jnp.ones((CHUNK,), jnp.float32)
    h = _layer(xp, W0, b0, rv0, het, z2, ones2)
    h = _layer(h, W1, b1, rv1, het, z2, ones2)
    return h[:N]


# speedup vs baseline: 22.7476x; 1.0429x over previous
import functools

import jax
import jax.numpy as jnp
from jax import lax
from jax.experimental import pallas as pl
from jax.experimental.pallas import tpu as pltpu
from jax.experimental.pallas import tpu_sc as plsc

N = 100000
NP = 100352
EH = 100000
EH_PAD = 102400
K = 16
NW = 32
EPT = EH_PAD // NW
CHUNK = 128
NCHUNK = EPT // CHUNK
TSLAB = NP // 16
ROWBLK = 2048

_MESH = plsc.VectorSubcoreMesh(core_axis_name="c", subcore_axis_name="s")
_SC_PARAMS = pltpu.CompilerParams(needs_layout_passes=False,
                                  use_tc_tiling_on_sc=False)


def _proj_body(x_ref, w_ref, rv_ref, m_ref, p_ref):
    xb = x_ref[...].astype(jnp.bfloat16)
    m_ref[...] = jnp.dot(xb, w_ref[...].astype(jnp.bfloat16),
                         preferred_element_type=jnp.float32)
    p_ref[...] = jnp.dot(xb, rv_ref[...].astype(jnp.bfloat16),
                         preferred_element_type=jnp.float32)


def _tc_project(hx, w, rv):
    din = hx.shape[1]
    dout = w.shape[1]
    return pl.pallas_call(
        _proj_body,
        grid=(NP // ROWBLK,),
        in_specs=[
            pl.BlockSpec((ROWBLK, din), lambda i: (i, 0)),
            pl.BlockSpec((din, dout), lambda i: (0, 0)),
            pl.BlockSpec((din, 1), lambda i: (0, 0)),
        ],
        out_specs=[
            pl.BlockSpec((ROWBLK, dout), lambda i: (i, 0)),
            pl.BlockSpec((ROWBLK, 1), lambda i: (i, 0)),
        ],
        out_shape=[
            jax.ShapeDtypeStruct((NP, dout), jnp.float32),
            jax.ShapeDtypeStruct((NP, 1), jnp.float32),
        ],
    )(hx, w, rv)


def _norm_body(dega_ref, degb_ref, m_ref, dinv_ref, mn_ref):
    deg = 1.0 + dega_ref[...] + degb_ref[...]
    dv = lax.rsqrt(deg)
    dvc = dv.reshape(ROWBLK, 1)
    dinv_ref[...] = jnp.broadcast_to(dvc, (ROWBLK, 16))
    mn_ref[...] = m_ref[...] * dvc


def _tc_norm(dega, degb, m):
    spec = pl.BlockSpec((ROWBLK, 16), lambda i: (i, 0))
    dspec = pl.BlockSpec((1, ROWBLK), lambda i: (0, i))
    return pl.pallas_call(
        _norm_body,
        grid=(NP // ROWBLK,),
        in_specs=[dspec, dspec, spec],
        out_specs=[spec, spec],
        out_shape=[
            jax.ShapeDtypeStruct((NP, 16), jnp.float32),
            jax.ShapeDtypeStruct((NP, 16), jnp.float32),
        ],
    )(dega, degb, m)


def _outproj_body(dinv_ref, agga_ref, aggb_ref, m_ref, b_ref, w_ref, rv_ref,
                  m2_ref, p2_ref):
    dv = dinv_ref[...]
    agg = agga_ref[...] + aggb_ref[...]
    h = jnp.maximum(dv * agg + (dv * dv) * m_ref[...] + b_ref[...], 0.0)
    hb = h.astype(jnp.bfloat16)
    m2_ref[...] = jnp.dot(hb, w_ref[...].astype(jnp.bfloat16),
                          preferred_element_type=jnp.float32)
    p2_ref[...] = jnp.dot(hb, rv_ref[...].astype(jnp.bfloat16),
                          preferred_element_type=jnp.float32)


def _tc_outproj(dinv, agga, aggb, m, b2d, w, rv):
    spec = pl.BlockSpec((ROWBLK, 16), lambda i: (i, 0))
    return pl.pallas_call(
        _outproj_body,
        grid=(NP // ROWBLK,),
        in_specs=[spec, spec, spec, spec,
                  pl.BlockSpec((1, 16), lambda i: (0, 0)),
                  pl.BlockSpec((16, 16), lambda i: (0, 0)),
                  pl.BlockSpec((16, 1), lambda i: (0, 0))],
        out_specs=[spec, pl.BlockSpec((ROWBLK, 1), lambda i: (i, 0))],
        out_shape=[
            jax.ShapeDtypeStruct((NP, 16), jnp.float32),
            jax.ShapeDtypeStruct((NP, 1), jnp.float32),
        ],
    )(dinv, agga, aggb, m, b2d, w, rv)


def _out_body(dinv_ref, agga_ref, aggb_ref, m_ref, b_ref, o_ref):
    dv = dinv_ref[...]
    agg = agga_ref[...] + aggb_ref[...]
    o = dv * agg + (dv * dv) * m_ref[...] + b_ref[...]
    o_ref[...] = jnp.maximum(o, 0.0)


def _tc_out(dinv, agga, aggb, m, b2d):
    spec = pl.BlockSpec((ROWBLK, 16), lambda i: (i, 0))
    return pl.pallas_call(
        _out_body,
        grid=(NP // ROWBLK,),
        in_specs=[spec, spec, spec, spec,
                  pl.BlockSpec((1, 16), lambda i: (0, 0))],
        out_specs=spec,
        out_shape=jax.ShapeDtypeStruct((NP, 16), jnp.float32),
    )(dinv, agga, aggb, m, b2d)


@functools.partial(
    pl.kernel,
    out_type=[
        jax.ShapeDtypeStruct((EH_PAD,), jnp.int32),
        jax.ShapeDtypeStruct((EH_PAD,), jnp.int32),
        jax.ShapeDtypeStruct((2, NP), jnp.float32),
    ],
    mesh=_MESH,
    compiler_params=_SC_PARAMS,
    scratch_types=[
        pltpu.VMEM((NP,), jnp.float32),
        pltpu.VMEM((K, CHUNK), jnp.int32),
        pltpu.VMEM((CHUNK,), jnp.int32),
        pltpu.VMEM((CHUNK,), jnp.int32),
        pltpu.VMEM((CHUNK,), jnp.float32),
        pltpu.VMEM_SHARED((NP,), jnp.float32),
        pltpu.SemaphoreType.DMA,
    ],
)
def _sc_edges(p_hbm, het_hbm, z1_hbm, ones_hbm,
              se_hbm, ie_hbm, deg_hbm,
              p_v, he_v, se_v, ie_v, ones_v, deg_sp, sem):
    ci = lax.axis_index("c")
    si = lax.axis_index("s")
    wid = si * 2 + ci

    pltpu.sync_copy(p_hbm, p_v)
    pltpu.sync_copy(ones_hbm, ones_v)
    pltpu.sync_copy(z1_hbm.at[pl.ds(si * TSLAB, TSLAB)],
                    deg_sp.at[pl.ds(si * TSLAB, TSLAB)])
    plsc.subcore_barrier()

    def chunk_body(c, _):
        base = wid * EPT + c * CHUNK
        pltpu.sync_copy(het_hbm.at[:, pl.ds(base, CHUNK)], he_v)

        def group_body(j, _):
            he0 = he_v[0, pl.ds(j * 16, 16)]
            pe0 = plsc.load_gather(p_v, [he0])
            cmax = pe0
            cmin = pe0
            se = he0
            ie = he0
            for k in range(1, K):
                hek = he_v[k, pl.ds(j * 16, 16)]
                pek = plsc.load_gather(p_v, [hek])
                gt = pek > cmax
                lt = pek < cmin
                se = jnp.where(gt, hek, se)
                cmax = jnp.where(gt, pek, cmax)
                ie = jnp.where(lt, hek, ie)
                cmin = jnp.where(lt, pek, cmin)
            se_v[pl.ds(j * 16, 16)] = se
            ie_v[pl.ds(j * 16, 16)] = ie
            return 0

        lax.fori_loop(0, CHUNK // 16, group_body, 0)
        pltpu.sync_copy(se_v, se_hbm.at[pl.ds(base, CHUNK)])
        pltpu.sync_copy(ie_v, ie_hbm.at[pl.ds(base, CHUNK)])
        pltpu.sync_copy(ones_v, deg_sp.at[se_v], add=True)
        pltpu.sync_copy(ones_v, deg_sp.at[ie_v], add=True)
        return 0

    lax.fori_loop(0, NCHUNK, chunk_body, 0)
    plsc.subcore_barrier()
    pltpu.sync_copy(deg_sp.at[pl.ds(si * TSLAB, TSLAB)],
                    deg_hbm.at[ci, pl.ds(si * TSLAB, TSLAB)])


@functools.partial(
    pl.kernel,
    out_type=jax.ShapeDtypeStruct((2, NP, 16), jnp.float32),
    mesh=_MESH,
    compiler_params=_SC_PARAMS,
    scratch_types=[
        pltpu.VMEM((CHUNK,), jnp.int32),
        pltpu.VMEM((CHUNK,), jnp.int32),
        pltpu.VMEM((CHUNK, 16), jnp.float32),
        pltpu.VMEM((CHUNK, 16), jnp.float32),
        pltpu.VMEM_SHARED((NP, 16), jnp.float32),
        pltpu.SemaphoreType.DMA,
    ],
)
def _sc_agg(se_hbm, ie_hbm, mn_hbm, z2_hbm,
            agg_hbm,
            se_v, ie_v, rows_a, rows_b, agg_sp, sem):
    ci = lax.axis_index("c")
    si = lax.axis_index("s")
    wid = si * 2 + ci

    pltpu.sync_copy(z2_hbm.at[pl.ds(si * TSLAB, TSLAB)],
                    agg_sp.at[pl.ds(si * TSLAB, TSLAB)])
    plsc.subcore_barrier()

    def chunk_body(c, _):
        base = wid * EPT + c * CHUNK
        pltpu.sync_copy(se_hbm.at[pl.ds(base, CHUNK)], se_v)
        pltpu.sync_copy(ie_hbm.at[pl.ds(base, CHUNK)], ie_v)
        ca = pltpu.async_copy(mn_hbm.at[ie_v], rows_a, sem)
        cb = pltpu.async_copy(mn_hbm.at[se_v], rows_b, sem)
        ca.wait()
        cb.wait()
        pltpu.sync_copy(rows_a, agg_sp.at[se_v], add=True)
        pltpu.sync_copy(rows_b, agg_sp.at[ie_v], add=True)
        return 0

    lax.fori_loop(0, NCHUNK, chunk_body, 0)
    plsc.subcore_barrier()
    pltpu.sync_copy(agg_sp.at[pl.ds(si * TSLAB, TSLAB)],
                    agg_hbm.at[ci, pl.ds(si * TSLAB, TSLAB)])


def _edges_norm_agg(p, m, het, z2, ones2):
    se, ie, deg2 = _sc_edges(p, het, z2[:, 0], ones2)
    dinv, mn = _tc_norm(deg2[0].reshape(1, NP), deg2[1].reshape(1, NP), m)
    agg2 = _sc_agg(se, ie, mn, z2)
    return dinv, agg2


def kernel(x, W0, b0, W1, b1, hyperedges):
    rv_key = jax.random.key(1)
    rv0 = jax.random.uniform(jax.random.fold_in(rv_key, 0), (128,),
                             dtype=jnp.float32)
    rv1 = jax.random.uniform(jax.random.fold_in(rv_key, 1), (16,),
                             dtype=jnp.float32)
    xp = jnp.pad(x, ((0, NP - N), (0, 0)))
    het = jnp.pad(hyperedges.astype(jnp.int32), ((0, EH_PAD - EH), (0, 0)),
                  constant_values=N).T
    z2 = jnp.zeros((NP, 16), jnp.float32)
    ones2 = jnp.ones((CHUNK,), jnp.float32)

    m1, p1 = _tc_project(xp, W0, rv0[:, None])
    dinv1, agg1 = _edges_norm_agg(p1.reshape(NP), m1, het, z2, ones2)
    m2, p2 = _tc_outproj(dinv1, agg1[0], agg1[1], m1, b0.reshape(1, 16),
                         W1, rv1[:, None])
    dinv2, agg2 = _edges_norm_agg(p2.reshape(NP), m2, het, z2, ones2)
    out = _tc_out(dinv2, agg2[0], agg2[1], m2, b1.reshape(1, 16))
    return out[:N]
